# baseline (speedup 1.0000x reference)
import functools

import jax
import jax.numpy as jnp
from jax import lax
from jax.experimental import pallas as pl
from jax.experimental.pallas import tpu as pltpu
from jax.experimental.pallas import tpu_sc as plsc

_ALPHA = 0.9
_NLAYERS = 3
_N = 100000
_E = 3200000
_C = 16

_NC = 2
_NS = 16
_NW = _NC * _NS

_CH = 128
_IB = 8
_NCHUNKS = _E // _CH
_TB = _NCHUNKS // _IB
_TBQ, _TBR = divmod(_TB, _NW)

_SPR = 100096
_NZCH = _SPR // _CH

_RCH = 400
_NRCHUNKS = _N // _RCH

_mesh = plsc.VectorSubcoreMesh(core_axis_name="c", subcore_axis_name="s")
_params = pltpu.CompilerParams(use_tc_tiling_on_sc=False)


def _fill(ref, val):
    def body(j, carry):
        ref[j, :] = jnp.full((_C,), val, jnp.float32)
        return carry
    lax.fori_loop(0, ref.shape[0], body, 0)


def _zero_spmem(sp, zbuf, sem, dummy_hbm):
    sid = lax.axis_index("s")
    nchunks = jnp.where(sid < (_NZCH % _NS), _NZCH // _NS + 1, _NZCH // _NS)
    def body(j, carry):
        pltpu.async_copy(zbuf, sp.at[pl.ds((sid + j * _NS) * _CH, _CH), :],
                         sem)
        return carry
    lax.fori_loop(0, nchunks, body, 0)
    def dbody(j, carry):
        pltpu.make_async_copy(dummy_hbm, zbuf, sem).wait()
        return carry
    lax.fori_loop(0, nchunks, dbody, 0)


def _drain_spmem(sp, hbm, cid, sid, sem):
    nchunks = jnp.where(sid < (_NRCHUNKS % _NS), _NRCHUNKS // _NS + 1,
                        _NRCHUNKS // _NS)
    def body(j, carry):
        base = (sid + j * _NS) * _RCH
        pltpu.async_copy(sp.at[pl.ds(base, _RCH), :],
                         hbm.at[pl.ds(cid * _N + base, _RCH), :], sem)
        return carry
    lax.fori_loop(0, nchunks, body, 0)
    def dbody(j, carry):
        pltpu.make_async_copy(hbm.at[pl.ds(0, _RCH), :],
                              sp.at[pl.ds(0, _RCH), :], sem).wait()
        return carry
    lax.fori_loop(0, nchunks, dbody, 0)


def _rsqrt16(d):
    i = lax.bitcast_convert_type(d, jnp.int32)
    i = jnp.int32(0x5F3759DF) - (i >> 1)
    r = lax.bitcast_convert_type(i, jnp.float32)
    for _ in range(3):
        r = r * (1.5 - 0.5 * d * r * r)
    return r


def _worker_chunks(w):
    return jnp.where(w < (_NRCHUNKS % _NW), _NRCHUNKS // _NW + 1,
                     _NRCHUNKS // _NW)


def _worker_batches(w):
    nblk = jnp.where(w < _TBR, _TBQ + 1, _TBQ)
    start = w * _TBQ + jnp.minimum(w, _TBR)
    return nblk, start


def _slot_drain(sem, dummy_hbm, ref):
    pltpu.make_async_copy(dummy_hbm, ref, sem).wait()


@functools.partial(
    pl.kernel,
    out_type=jax.ShapeDtypeStruct((_NC * _N, _C), jnp.float32),
    mesh=_mesh,
    compiler_params=_params,
    scratch_types=[
        pltpu.VMEM((2, _IB, _CH), jnp.int32),
        pltpu.VMEM((_CH, _C), jnp.float32),
        pltpu.VMEM_SHARED((_SPR, _C), jnp.float32),
    ] + [pltpu.SemaphoreType.DMA] * (_IB + 2),
)
def _deg_kernel(dst_hbm, deg_hbm, idx_v, buf, deg_sp, *sems):
    ssem = sems[:_IB]
    zsem = sems[_IB]
    isem = sems[_IB + 1]
    cid = lax.axis_index("c")
    sid = lax.axis_index("s")
    w = cid * _NS + sid
    _fill(buf, 0.0)
    _zero_spmem(deg_sp, buf, zsem, deg_hbm.at[pl.ds(0, _CH), :])
    _fill(buf, 1.0)
    plsc.subcore_barrier()

    nblk, start = _worker_batches(w)
    pltpu.async_copy(dst_hbm.at[pl.ds(start * _IB, _IB), :], idx_v.at[0],
                     isem)

    def blk_body(blk, carry):
        p = lax.rem(blk, 2)
        _slot_drain(isem, dst_hbm.at[pl.ds(0, _IB), :], idx_v.at[p])
        for b in range(_IB):
            @pl.when(blk > 0)
            def _(b=b):
                _slot_drain(ssem[b], deg_hbm.at[pl.ds(0, _CH), :], buf)
        @pl.when(blk + 1 < nblk)
        def _():
            pltpu.async_copy(
                dst_hbm.at[pl.ds((start + blk + 1) * _IB, _IB), :],
                idx_v.at[1 - p], isem)
        for b in range(_IB):
            pltpu.async_copy(buf, deg_sp.at[idx_v.at[p].at[b]], ssem[b],
                             add=True)
        return carry
    lax.fori_loop(0, nblk, blk_body, 0)

    for b in range(_IB):
        _slot_drain(ssem[b], deg_hbm.at[pl.ds(0, _CH), :], buf)

    plsc.subcore_barrier()
    _drain_spmem(deg_sp, deg_hbm, cid, sid, zsem)


@functools.partial(
    pl.kernel,
    out_type=jax.ShapeDtypeStruct((_NC * _N, _C), jnp.float32),
    mesh=_mesh,
    compiler_params=_params,
    scratch_types=[
        pltpu.VMEM((2, _IB, _CH), jnp.int32),
        pltpu.VMEM((2, _IB, _CH), jnp.int32),
        pltpu.VMEM((_IB, _CH, _C), jnp.float32),
        pltpu.VMEM((_CH, _C), jnp.float32),
        pltpu.VMEM_SHARED((_SPR, _C), jnp.float32),
    ] + [pltpu.SemaphoreType.DMA] * (2 * _IB + 2),
)
def _edge_kernel(src_hbm, dst_hbm, h_hbm, agg_hbm,
                 sidx, didx, rows, zbuf, agg_sp, *sems):
    gsem = sems[:_IB]
    ssem = sems[_IB:2 * _IB]
    zsem = sems[2 * _IB]
    isem = sems[2 * _IB + 1]
    cid = lax.axis_index("c")
    sid = lax.axis_index("s")
    w = cid * _NS + sid
    _fill(zbuf, 0.0)
    _zero_spmem(agg_sp, zbuf, zsem, agg_hbm.at[pl.ds(0, _CH), :])
    plsc.subcore_barrier()

    nblk, start = _worker_batches(w)
    pltpu.async_copy(src_hbm.at[pl.ds(start * _IB, _IB), :], sidx.at[0],
                     isem)
    pltpu.async_copy(dst_hbm.at[pl.ds(start * _IB, _IB), :], didx.at[0],
                     isem)

    def blk_body(blk, carry):
        p = lax.rem(blk, 2)
        _slot_drain(isem, src_hbm.at[pl.ds(0, _IB), :], sidx.at[p])
        _slot_drain(isem, dst_hbm.at[pl.ds(0, _IB), :], didx.at[p])
        gh = []
        for b in range(_IB):
            @pl.when(blk > 0)
            def _(b=b):
                _slot_drain(ssem[b], agg_hbm.at[pl.ds(0, _CH), :],
                            rows.at[b])
            gh.append(pltpu.async_copy(h_hbm.at[sidx.at[p].at[b]],
                                       rows.at[b], gsem[b]))
        @pl.when(blk + 1 < nblk)
        def _():
            nbase = (start + blk + 1) * _IB
            pltpu.async_copy(src_hbm.at[pl.ds(nbase, _IB), :],
                             sidx.at[1 - p], isem)
            pltpu.async_copy(dst_hbm.at[pl.ds(nbase, _IB), :],
                             didx.at[1 - p], isem)
        for b in range(_IB):
            gh[b].wait()
            pltpu.async_copy(rows.at[b], agg_sp.at[didx.at[p].at[b]],
                             ssem[b], add=True)
        return carry
    lax.fori_loop(0, nblk, blk_body, 0)

    for b in range(_IB):
        _slot_drain(ssem[b], agg_hbm.at[pl.ds(0, _CH), :], rows.at[b])

    plsc.subcore_barrier()
    _drain_spmem(agg_sp, agg_hbm, cid, sid, zsem)


_TBR_TC = 4000
_TG = _N // _TBR_TC


def _tc_blk(off):
    return pl.BlockSpec((_TBR_TC, _C), lambda i, off=off: (i + off, 0))


def _init_tc_body(d0_ref, d1_ref, y_ref, m_ref, n_ref, h_ref, l_ref):
    d = jnp.maximum(d0_ref[...] + d1_ref[...], 1.0)
    r = lax.rsqrt(d)
    ym = y_ref[...] * m_ref[...]
    n_ref[...] = r
    h_ref[...] = r * ym
    l_ref[...] = (1.0 - _ALPHA) * ym


def _init_kernel(degp, y, mrows):
    return pl.pallas_call(
        _init_tc_body,
        grid=(_TG,),
        in_specs=[_tc_blk(0), _tc_blk(_TG), _tc_blk(0), _tc_blk(0)],
        out_specs=[_tc_blk(0), _tc_blk(0), _tc_blk(0)],
        out_shape=(
            jax.ShapeDtypeStruct((_N, _C), jnp.float32),
            jax.ShapeDtypeStruct((_N, _C), jnp.float32),
            jax.ShapeDtypeStruct((_N, _C), jnp.float32),
        ),
    )(degp, degp, y, mrows)


def _combine_tc_body(a0_ref, a1_ref, n_ref, l_ref, o_ref, h_ref):
    a = a0_ref[...] + a1_ref[...]
    o = l_ref[...] + _ALPHA * (n_ref[...] * a)
    o = jnp.minimum(jnp.maximum(o, 0.0), 1.0)
    o_ref[...] = o
    h_ref[...] = n_ref[...] * o


def _combine_kernel(aggp, norm, last):
    return pl.pallas_call(
        _combine_tc_body,
        grid=(_TG,),
        in_specs=[_tc_blk(0), _tc_blk(_TG), _tc_blk(0), _tc_blk(0)],
        out_specs=[_tc_blk(0), _tc_blk(0)],
        out_shape=(
            jax.ShapeDtypeStruct((_N, _C), jnp.float32),
            jax.ShapeDtypeStruct((_N, _C), jnp.float32),
        ),
    )(aggp, aggp, norm, last)


def kernel(y, edge_index, mask):
    src2d = edge_index[0].reshape(_NCHUNKS, _CH)
    dst2d = edge_index[1].reshape(_NCHUNKS, _CH)
    mrows = jnp.broadcast_to(mask.astype(jnp.float32)[:, None], (_N, _C))

    degp = _deg_kernel(dst2d)
    norm, h, last = _init_kernel(degp, y, mrows)
    out = None
    for _ in range(_NLAYERS):
        aggp = _edge_kernel(src2d, dst2d, h)
        out, h = _combine_kernel(aggp, norm, last)
    return out


# speedup vs baseline: 1.8317x; 1.8317x over previous
import functools

import jax
import jax.numpy as jnp
from jax import lax
from jax.experimental import pallas as pl
from jax.experimental.pallas import tpu as pltpu
from jax.experimental.pallas import tpu_sc as plsc

_ALPHA = 0.9
_NLAYERS = 3
_N = 100000
_E = 3200000
_C = 16

_NC = 2
_NS = 16
_NW = _NC * _NS

_CH = 128
_IB = 10
_NCHUNKS = _E // _CH
_TB = _NCHUNKS // _IB
_TBQ, _TBR = divmod(_TB, _NW)

_SPR = 100096
_NZCH = _SPR // _CH

_RCH = 400
_NRCHUNKS = _N // _RCH

_mesh = plsc.VectorSubcoreMesh(core_axis_name="c", subcore_axis_name="s")
_params = pltpu.CompilerParams(use_tc_tiling_on_sc=False)


def _fill(ref, val):
    def body(j, carry):
        ref[j, :] = jnp.full((_C,), val, jnp.float32)
        return carry
    lax.fori_loop(0, ref.shape[0], body, 0)


def _zero_spmem(sp, zbuf, sem, dummy_hbm):
    sid = lax.axis_index("s")
    nchunks = jnp.where(sid < (_NZCH % _NS), _NZCH // _NS + 1, _NZCH // _NS)
    def body(j, carry):
        pltpu.async_copy(zbuf, sp.at[pl.ds((sid + j * _NS) * _CH, _CH), :],
                         sem)
        return carry
    lax.fori_loop(0, nchunks, body, 0)
    def dbody(j, carry):
        pltpu.make_async_copy(dummy_hbm, zbuf, sem).wait()
        return carry
    lax.fori_loop(0, nchunks, dbody, 0)


def _drain_spmem(sp, hbm, cid, sid, sem):
    nchunks = jnp.where(sid < (_NRCHUNKS % _NS), _NRCHUNKS // _NS + 1,
                        _NRCHUNKS // _NS)
    def body(j, carry):
        base = (sid + j * _NS) * _RCH
        pltpu.async_copy(sp.at[pl.ds(base, _RCH), :],
                         hbm.at[pl.ds(cid * _N + base, _RCH), :], sem)
        return carry
    lax.fori_loop(0, nchunks, body, 0)
    def dbody(j, carry):
        pltpu.make_async_copy(hbm.at[pl.ds(0, _RCH), :],
                              sp.at[pl.ds(0, _RCH), :], sem).wait()
        return carry
    lax.fori_loop(0, nchunks, dbody, 0)


def _rsqrt16(d):
    i = lax.bitcast_convert_type(d, jnp.int32)
    i = jnp.int32(0x5F3759DF) - (i >> 1)
    r = lax.bitcast_convert_type(i, jnp.float32)
    for _ in range(3):
        r = r * (1.5 - 0.5 * d * r * r)
    return r


def _worker_chunks(w):
    return jnp.where(w < (_NRCHUNKS % _NW), _NRCHUNKS // _NW + 1,
                     _NRCHUNKS // _NW)


def _worker_batches(w):
    nblk = jnp.where(w < _TBR, _TBQ + 1, _TBQ)
    start = w * _TBQ + jnp.minimum(w, _TBR)
    return nblk, start


def _slot_drain(sem, dummy_hbm, ref):
    pltpu.make_async_copy(dummy_hbm, ref, sem).wait()


@functools.partial(
    pl.kernel,
    out_type=jax.ShapeDtypeStruct((_NC * _N, _C), jnp.float32),
    mesh=_mesh,
    compiler_params=_params,
    scratch_types=[
        pltpu.VMEM((2, _IB, _CH), jnp.int32),
        pltpu.VMEM((_CH, _C), jnp.float32),
        pltpu.VMEM_SHARED((_SPR, _C), jnp.float32),
    ] + [pltpu.SemaphoreType.DMA] * (_IB + 2),
)
def _deg_kernel(dst_hbm, deg_hbm, idx_v, buf, deg_sp, *sems):
    ssem = sems[:_IB]
    zsem = sems[_IB]
    isem = sems[_IB + 1]
    cid = lax.axis_index("c")
    sid = lax.axis_index("s")
    w = cid * _NS + sid
    _fill(buf, 0.0)
    _zero_spmem(deg_sp, buf, zsem, deg_hbm.at[pl.ds(0, _CH), :])
    _fill(buf, 1.0)
    plsc.subcore_barrier()

    nblk, start = _worker_batches(w)
    pltpu.async_copy(dst_hbm.at[pl.ds(start * _IB, _IB), :], idx_v.at[0],
                     isem)

    def blk_body(blk, carry):
        p = lax.rem(blk, 2)
        _slot_drain(isem, dst_hbm.at[pl.ds(0, _IB), :], idx_v.at[p])
        for b in range(_IB):
            @pl.when(blk > 0)
            def _(b=b):
                _slot_drain(ssem[b], deg_hbm.at[pl.ds(0, _CH), :], buf)
        @pl.when(blk + 1 < nblk)
        def _():
            pltpu.async_copy(
                dst_hbm.at[pl.ds((start + blk + 1) * _IB, _IB), :],
                idx_v.at[1 - p], isem)
        for b in range(_IB):
            pltpu.async_copy(buf, deg_sp.at[idx_v.at[p].at[b]], ssem[b],
                             add=True)
        return carry
    lax.fori_loop(0, nblk, blk_body, 0)

    for b in range(_IB):
        _slot_drain(ssem[b], deg_hbm.at[pl.ds(0, _CH), :], buf)

    plsc.subcore_barrier()
    _drain_spmem(deg_sp, deg_hbm, cid, sid, zsem)


@functools.partial(
    pl.kernel,
    out_type=jax.ShapeDtypeStruct((_NC * _N, _C), jnp.float32),
    mesh=_mesh,
    compiler_params=_params,
    scratch_types=[
        pltpu.VMEM((2, _IB, _CH), jnp.int32),
        pltpu.VMEM((2, _IB, _CH), jnp.int32),
        pltpu.VMEM((_IB, _CH, _C), jnp.float32),
        pltpu.VMEM((_CH, _C), jnp.float32),
        pltpu.VMEM_SHARED((_SPR, _C), jnp.float32),
    ] + [pltpu.SemaphoreType.DMA] * (2 * _IB + 2),
)
def _edge_kernel(src_hbm, dst_hbm, h_hbm, agg_hbm,
                 sidx, didx, rows, zbuf, agg_sp, *sems):
    gsem = sems[:_IB]
    ssem = sems[_IB:2 * _IB]
    zsem = sems[2 * _IB]
    isem = sems[2 * _IB + 1]
    cid = lax.axis_index("c")
    sid = lax.axis_index("s")
    w = cid * _NS + sid
    _fill(zbuf, 0.0)
    _zero_spmem(agg_sp, zbuf, zsem, agg_hbm.at[pl.ds(0, _CH), :])
    plsc.subcore_barrier()

    nblk, start = _worker_batches(w)
    pltpu.async_copy(src_hbm.at[pl.ds(start * _IB, _IB), :], sidx.at[0],
                     isem)
    pltpu.async_copy(dst_hbm.at[pl.ds(start * _IB, _IB), :], didx.at[0],
                     isem)

    def blk_body(blk, carry):
        p = lax.rem(blk, 2)
        _slot_drain(isem, src_hbm.at[pl.ds(0, _IB), :], sidx.at[p])
        _slot_drain(isem, dst_hbm.at[pl.ds(0, _IB), :], didx.at[p])
        gh = []
        for b in range(_IB):
            @pl.when(blk > 0)
            def _(b=b):
                _slot_drain(ssem[b], agg_hbm.at[pl.ds(0, _CH), :],
                            rows.at[b])
            gh.append(pltpu.async_copy(h_hbm.at[sidx.at[p].at[b]],
                                       rows.at[b], gsem[b]))
        @pl.when(blk + 1 < nblk)
        def _():
            nbase = (start + blk + 1) * _IB
            pltpu.async_copy(src_hbm.at[pl.ds(nbase, _IB), :],
                             sidx.at[1 - p], isem)
            pltpu.async_copy(dst_hbm.at[pl.ds(nbase, _IB), :],
                             didx.at[1 - p], isem)
        for b in range(_IB):
            gh[b].wait()
            pltpu.async_copy(rows.at[b], agg_sp.at[didx.at[p].at[b]],
                             ssem[b], add=True)
        return carry
    lax.fori_loop(0, nblk, blk_body, 0)

    for b in range(_IB):
        _slot_drain(ssem[b], agg_hbm.at[pl.ds(0, _CH), :], rows.at[b])

    plsc.subcore_barrier()
    _drain_spmem(agg_sp, agg_hbm, cid, sid, zsem)


@functools.partial(
    pl.kernel,
    out_type=(
        jax.ShapeDtypeStruct((_N, _C), jnp.float32),
        jax.ShapeDtypeStruct((_N, _C), jnp.float32),
        jax.ShapeDtypeStruct((_N, _C), jnp.float32),
    ),
    mesh=_mesh,
    compiler_params=_params,
    scratch_types=[
        pltpu.VMEM((2, _RCH, _C), jnp.float32),
        pltpu.VMEM((2, _RCH, _C), jnp.float32),
        pltpu.VMEM((2, _RCH, _C), jnp.float32),
        pltpu.VMEM((2, _RCH, _C), jnp.float32),
        pltpu.VMEM((2, _RCH, _C), jnp.float32),
        pltpu.VMEM((2, _RCH, _C), jnp.float32),
        pltpu.VMEM((2, _RCH, _C), jnp.float32),
        pltpu.SemaphoreType.DMA,
        pltpu.SemaphoreType.DMA,
        pltpu.SemaphoreType.DMA,
    ],
)
def _init_kernel(deg_hbm, y_hbm, m_hbm, norm_hbm, h_hbm, last_hbm,
                 d0, d1, yb, mb, nb, hb, lb, isem, osem0, osem1):
    osem = (osem0, osem1)
    cid = lax.axis_index("c")
    sid = lax.axis_index("s")
    w = cid * _NS + sid
    nch = _worker_chunks(w)

    def load(k, p):
        base = (w + k * _NW) * _RCH
        pltpu.async_copy(deg_hbm.at[pl.ds(base, _RCH), :], d0.at[p], isem)
        pltpu.async_copy(deg_hbm.at[pl.ds(_N + base, _RCH), :], d1.at[p],
                         isem)
        pltpu.async_copy(y_hbm.at[pl.ds(base, _RCH), :], yb.at[p], isem)
        pltpu.async_copy(m_hbm.at[pl.ds(base, _RCH), :], mb.at[p], isem)

    load(0, 0)

    for k in range(_NRCHUNKS // _NW + 1):
        p = k % 2

        @pl.when(k < nch)
        def _(k=k, p=p):
            base = (w + k * _NW) * _RCH
            for ref in (d0, d1, yb, mb):
                _slot_drain(isem, y_hbm.at[pl.ds(0, _RCH), :], ref.at[p])
            @pl.when(k + 1 < nch)
            def _():
                load(k + 1, 1 - p)
            if k > 1:
                for ref in (nb, hb, lb):
                    _slot_drain(osem[p], y_hbm.at[pl.ds(0, _RCH), :],
                                ref.at[p])

            def row_body(i, c2):
                for off in (0, _RCH // 4, _RCH // 2, 3 * _RCH // 4):
                    ii = i + off
                    d = d0[p, ii, :] + d1[p, ii, :]
                    d = jnp.maximum(d, 1.0)
                    r = _rsqrt16(d)
                    ym = yb[p, ii, :] * mb[p, ii, :]
                    nb[p, ii, :] = r
                    hb[p, ii, :] = r * ym
                    lb[p, ii, :] = (1.0 - _ALPHA) * ym
                return c2
            lax.fori_loop(0, _RCH // 4, row_body, 0)

            pltpu.async_copy(nb.at[p], norm_hbm.at[pl.ds(base, _RCH), :],
                             osem[p])
            pltpu.async_copy(hb.at[p], h_hbm.at[pl.ds(base, _RCH), :],
                             osem[p])
            pltpu.async_copy(lb.at[p], last_hbm.at[pl.ds(base, _RCH), :],
                             osem[p])

    for p in (0, 1):
        for ref in (nb, hb, lb):
            _slot_drain(osem[p], y_hbm.at[pl.ds(0, _RCH), :], ref.at[p])


@functools.partial(
    pl.kernel,
    out_type=(
        jax.ShapeDtypeStruct((_N, _C), jnp.float32),
        jax.ShapeDtypeStruct((_N, _C), jnp.float32),
    ),
    mesh=_mesh,
    compiler_params=_params,
    scratch_types=[
        pltpu.VMEM((2, _RCH, _C), jnp.float32),
        pltpu.VMEM((2, _RCH, _C), jnp.float32),
        pltpu.VMEM((2, _RCH, _C), jnp.float32),
        pltpu.VMEM((2, _RCH, _C), jnp.float32),
        pltpu.VMEM((2, _RCH, _C), jnp.float32),
        pltpu.VMEM((2, _RCH, _C), jnp.float32),
        pltpu.SemaphoreType.DMA,
        pltpu.SemaphoreType.DMA,
        pltpu.SemaphoreType.DMA,
    ],
)
def _combine_kernel(agg_hbm, norm_hbm, last_hbm, out_hbm, h_hbm,
                    a0, a1, nb, lb, ob, hb, isem, osem0, osem1):
    osem = (osem0, osem1)
    cid = lax.axis_index("c")
    sid = lax.axis_index("s")
    w = cid * _NS + sid
    nch = _worker_chunks(w)

    def load(k, p):
        base = (w + k * _NW) * _RCH
        pltpu.async_copy(agg_hbm.at[pl.ds(base, _RCH), :], a0.at[p], isem)
        pltpu.async_copy(agg_hbm.at[pl.ds(_N + base, _RCH), :], a1.at[p],
                         isem)
        pltpu.async_copy(norm_hbm.at[pl.ds(base, _RCH), :], nb.at[p], isem)
        pltpu.async_copy(last_hbm.at[pl.ds(base, _RCH), :], lb.at[p], isem)

    load(0, 0)

    for k in range(_NRCHUNKS // _NW + 1):
        p = k % 2

        @pl.when(k < nch)
        def _(k=k, p=p):
            base = (w + k * _NW) * _RCH
            for ref in (a0, a1, nb, lb):
                _slot_drain(isem, norm_hbm.at[pl.ds(0, _RCH), :], ref.at[p])
            @pl.when(k + 1 < nch)
            def _():
                load(k + 1, 1 - p)
            if k > 1:
                for ref in (ob, hb):
                    _slot_drain(osem[p], norm_hbm.at[pl.ds(0, _RCH), :],
                                ref.at[p])

            def row_body(i, c2):
                for off in (0, _RCH // 4, _RCH // 2, 3 * _RCH // 4):
                    ii = i + off
                    a = a0[p, ii, :] + a1[p, ii, :]
                    o = lb[p, ii, :] + _ALPHA * (nb[p, ii, :] * a)
                    o = jnp.minimum(jnp.maximum(o, 0.0), 1.0)
                    ob[p, ii, :] = o
                    hb[p, ii, :] = nb[p, ii, :] * o
                return c2
            lax.fori_loop(0, _RCH // 4, row_body, 0)

            pltpu.async_copy(ob.at[p], out_hbm.at[pl.ds(base, _RCH), :],
                             osem[p])
            pltpu.async_copy(hb.at[p], h_hbm.at[pl.ds(base, _RCH), :],
                             osem[p])

    for p in (0, 1):
        for ref in (ob, hb):
            _slot_drain(osem[p], norm_hbm.at[pl.ds(0, _RCH), :], ref.at[p])


def kernel(y, edge_index, mask):
    src2d = edge_index[0].reshape(_NCHUNKS, _CH)
    dst2d = edge_index[1].reshape(_NCHUNKS, _CH)
    mrows = jnp.broadcast_to(mask.astype(jnp.float32)[:, None], (_N, _C))

    degp = _deg_kernel(dst2d)
    norm, h, last = _init_kernel(degp, y, mrows)
    out = None
    for _ in range(_NLAYERS):
        aggp = _edge_kernel(src2d, dst2d, h)
        out, h = _combine_kernel(aggp, norm, last)
    return out
